# baseline (speedup 1.0000x reference)
import functools

import jax
import jax.numpy as jnp
from jax import lax
from jax.experimental import pallas as pl
from jax.experimental.pallas import tpu as pltpu
from jax.experimental.pallas import tpu_sc as plsc

_MAX_LEN = 2048
_DIM = 1024
_NC = 2
_NS = 16
_NW = _NC * _NS
_ROWS_PER_W = _MAX_LEN // _NW
_TS_ROWS = 48
_SP_ROWS = _ROWS_PER_W - _TS_ROWS

_mesh = plsc.VectorSubcoreMesh(core_axis_name="c", subcore_axis_name="s")


@functools.partial(
    pl.kernel,
    mesh=_mesh,
    out_type=jax.ShapeDtypeStruct((_MAX_LEN, _DIM), jnp.float32),
    scratch_types=[
        pltpu.VMEM((_TS_ROWS, _DIM), jnp.float32),
        pltpu.VMEM_SHARED((_NS, _SP_ROWS, _DIM), jnp.float32),
        pltpu.SemaphoreType.DMA,
        pltpu.SemaphoreType.DMA,
        pltpu.SemaphoreType.DMA,
        pltpu.SemaphoreType.DMA,
    ],
)
def _pos_embed_copy(w_hbm, out_hbm, tbuf, sbuf, sem_ti, sem_to, sem_si, sem_so):
    sid = lax.axis_index("s")
    wid = sid * _NC + lax.axis_index("c")
    base = wid * _ROWS_PER_W
    th = _TS_ROWS // 2
    sh = _SP_ROWS // 2

    ts_in = [
        pltpu.make_async_copy(
            w_hbm.at[pl.ds(base + i * th, th)], tbuf.at[pl.ds(i * th, th)], sem_ti
        )
        for i in range(2)
    ]
    sp_in = [
        pltpu.make_async_copy(
            w_hbm.at[pl.ds(base + _TS_ROWS + i * sh, sh)],
            sbuf.at[sid, pl.ds(i * sh, sh)],
            sem_si,
        )
        for i in range(2)
    ]
    ts_out = [
        pltpu.make_async_copy(
            tbuf.at[pl.ds(i * th, th)], out_hbm.at[pl.ds(base + i * th, th)], sem_to
        )
        for i in range(2)
    ]
    sp_out = [
        pltpu.make_async_copy(
            sbuf.at[sid, pl.ds(i * sh, sh)],
            out_hbm.at[pl.ds(base + _TS_ROWS + i * sh, sh)],
            sem_so,
        )
        for i in range(2)
    ]

    ts_in[0].start()
    sp_in[0].start()
    ts_in[1].start()
    sp_in[1].start()
    ts_in[0].wait()
    ts_out[0].start()
    sp_in[0].wait()
    sp_out[0].start()
    ts_in[1].wait()
    ts_out[1].start()
    sp_in[1].wait()
    sp_out[1].start()
    ts_out[0].wait()
    ts_out[1].wait()
    sp_out[0].wait()
    sp_out[1].wait()


def kernel(x, W):
    del x
    return _pos_embed_copy(W)
